# baseline (speedup 1.0000x reference)
import numpy as np
import jax
import jax.numpy as jnp
from jax import lax
from jax.experimental import pallas as pl
from jax.experimental.pallas import tpu as pltpu
from jax.experimental.pallas import tpu_sc as plsc

B, T, H, V = 1024, 200, 128, 256
NC, NS = 2, 16
NW = NC * NS
BPW = B // NW
RB = 128
NBLK = T * (B // RB)
BLKW = NBLK // NW
NBUF = 4

_t = np.arange(NBLK, dtype=np.int64) // (B // RB)
_b0 = (np.arange(NBLK, dtype=np.int64) % (B // RB)) * RB
_GIDX = ((_b0[:, None] + np.arange(RB)[None, :]) * T + _t[:, None]).astype(
    np.int32
)
BLKP = 56
_GIDX = np.zeros((NW, BLKP, RB), np.int32)
_GIDX[:, :BLKW, :] = (
    ((_b0[:, None] + np.arange(RB)[None, :]) * T + _t[:, None])
    .astype(np.int32)
    .reshape(NW, BLKW, RB)
)


def _sc_body(gidx_hbm, idx_hbm, table_hbm, mel_hbm, out_hbm,
             gidx_v, idx_v, rows_v, bufs, rsems, wsems, gat_sem):
    wid = lax.axis_index("s") * NC + lax.axis_index("c")
    m0 = wid * BLKW

    pltpu.sync_copy(idx_hbm.at[pl.ds(wid * BPW, BPW)], idx_v)
    pltpu.async_copy(table_hbm.at[idx_v], rows_v, gat_sem).wait()
    emb_wr = pltpu.async_copy(
        rows_v, out_hbm.at[pl.ds(wid * BPW, BPW), :], gat_sem
    )

    pltpu.sync_copy(gidx_hbm.at[wid], gidx_v)

    rd = [None] * BLKW
    wr = [None] * BLKW

    def start_read(i):
        return pltpu.async_copy(
            mel_hbm.at[gidx_v.at[i]], bufs[i % NBUF], rsems[i % NBUF]
        )

    for i in range(NBUF):
        rd[i] = start_read(i)
    for i in range(BLKW):
        rd[i].wait()
        m = m0 + i
        row0 = (1 + m // (B // RB)) * B + (m % (B // RB)) * RB
        wr[i] = pltpu.async_copy(
            bufs[i % NBUF], out_hbm.at[pl.ds(row0, RB), :], wsems[i % NBUF]
        )
        if i >= 1 and i + NBUF - 1 < BLKW:
            wr[i - 1].wait()
            rd[i + NBUF - 1] = start_read(i + NBUF - 1)
    for j in range(BLKW - NBUF, BLKW):
        wr[j].wait()
    emb_wr.wait()


@jax.jit
def _run(gidx, idx, table, mel2d):
    mesh = plsc.VectorSubcoreMesh(
        core_axis_name="c", subcore_axis_name="s", num_cores=NC, num_subcores=NS
    )
    out2d = pl.kernel(
        _sc_body,
        out_type=jax.ShapeDtypeStruct(((T + 1) * B, H), jnp.float32),
        mesh=mesh,
        scratch_types=[
            pltpu.VMEM((BLKP, RB), jnp.int32),
            pltpu.VMEM((BPW,), jnp.int32),
            pltpu.VMEM((BPW, H), jnp.float32),
            [pltpu.VMEM((RB, H), jnp.float32)] * NBUF,
            [pltpu.SemaphoreType.DMA] * NBUF,
            [pltpu.SemaphoreType.DMA] * NBUF,
            pltpu.SemaphoreType.DMA,
        ],
        compiler_params=pltpu.CompilerParams(use_tc_tiling_on_sc=True),
    )(gidx, idx, table, mel2d)
    return out2d.reshape(T + 1, B, H).transpose(1, 0, 2)


def kernel(arranger_id, mel_db, table):
    idx = arranger_id.reshape(B).astype(jnp.int32)
    gidx = jnp.asarray(_GIDX)
    return _run(gidx, idx, table, mel_db.reshape(B * T, H))


# speedup vs baseline: 1.0150x; 1.0150x over previous
import numpy as np
import jax
import jax.numpy as jnp
from jax import lax
from jax.experimental import pallas as pl
from jax.experimental.pallas import tpu as pltpu
from jax.experimental.pallas import tpu_sc as plsc

B, T, H, V = 1024, 200, 128, 256
NC, NS = 2, 16
NW = NC * NS
BPW = B // NW
RB = 128
NBLK = T * (B // RB)
BLKW = NBLK // NW
NBUF = 6

_t = np.arange(NBLK, dtype=np.int64) // (B // RB)
_b0 = (np.arange(NBLK, dtype=np.int64) % (B // RB)) * RB
_GIDX = ((_b0[:, None] + np.arange(RB)[None, :]) * T + _t[:, None]).astype(
    np.int32
)
BLKP = 56
_GIDX = np.zeros((NW, BLKP, RB), np.int32)
_GIDX[:, :BLKW, :] = (
    ((_b0[:, None] + np.arange(RB)[None, :]) * T + _t[:, None])
    .astype(np.int32)
    .reshape(NW, BLKW, RB)
)


def _sc_body(gidx_hbm, idx_hbm, table_hbm, mel_hbm, out_hbm,
             gidx_v, idx_v, rows_v, bufs, rsems, wsems, gat_sem):
    wid = lax.axis_index("s") * NC + lax.axis_index("c")
    m0 = wid * BLKW

    pltpu.sync_copy(idx_hbm.at[pl.ds(wid * BPW, BPW)], idx_v)
    pltpu.async_copy(table_hbm.at[idx_v], rows_v, gat_sem).wait()
    emb_wr = pltpu.async_copy(
        rows_v, out_hbm.at[pl.ds(wid * BPW, BPW), :], gat_sem
    )

    pltpu.sync_copy(gidx_hbm.at[wid], gidx_v)

    rd = [None] * BLKW
    wr = [None] * BLKW

    def start_read(i):
        return pltpu.async_copy(
            mel_hbm.at[gidx_v.at[i]], bufs[i % NBUF], rsems[i % NBUF]
        )

    for i in range(NBUF):
        rd[i] = start_read(i)
    for i in range(BLKW):
        rd[i].wait()
        m = m0 + i
        row0 = (1 + m // (B // RB)) * B + (m % (B // RB)) * RB
        wr[i] = pltpu.async_copy(
            bufs[i % NBUF], out_hbm.at[pl.ds(row0, RB), :], wsems[i % NBUF]
        )
        if i >= 1 and i + NBUF - 1 < BLKW:
            wr[i - 1].wait()
            rd[i + NBUF - 1] = start_read(i + NBUF - 1)
    for j in range(BLKW - NBUF, BLKW):
        wr[j].wait()
    emb_wr.wait()


@jax.jit
def _run(gidx, idx, table, mel2d):
    mesh = plsc.VectorSubcoreMesh(
        core_axis_name="c", subcore_axis_name="s", num_cores=NC, num_subcores=NS
    )
    out2d = pl.kernel(
        _sc_body,
        out_type=jax.ShapeDtypeStruct(((T + 1) * B, H), jnp.float32),
        mesh=mesh,
        scratch_types=[
            pltpu.VMEM((BLKP, RB), jnp.int32),
            pltpu.VMEM((BPW,), jnp.int32),
            pltpu.VMEM((BPW, H), jnp.float32),
            [pltpu.VMEM((RB, H), jnp.float32)] * NBUF,
            [pltpu.SemaphoreType.DMA] * NBUF,
            [pltpu.SemaphoreType.DMA] * NBUF,
            pltpu.SemaphoreType.DMA,
        ],
        compiler_params=pltpu.CompilerParams(use_tc_tiling_on_sc=True),
    )(gidx, idx, table, mel2d)
    return out2d.reshape(T + 1, B, H).transpose(1, 0, 2)


def kernel(arranger_id, mel_db, table):
    idx = arranger_id.reshape(B).astype(jnp.int32)
    gidx = jnp.asarray(_GIDX)
    return _run(gidx, idx, table, mel_db.reshape(B * T, H))


# speedup vs baseline: 1.0157x; 1.0008x over previous
import numpy as np
import jax
import jax.numpy as jnp
from jax import lax
from jax.experimental import pallas as pl
from jax.experimental.pallas import tpu as pltpu
from jax.experimental.pallas import tpu_sc as plsc

B, T, H, V = 1024, 200, 128, 256
NC, NS = 2, 16
NW = NC * NS
BPW = B // NW
RB = 128
NBLK = T * (B // RB)
BLKW = NBLK // NW
NBUF = 7

_t = np.arange(NBLK, dtype=np.int64) // (B // RB)
_b0 = (np.arange(NBLK, dtype=np.int64) % (B // RB)) * RB
_GIDX = ((_b0[:, None] + np.arange(RB)[None, :]) * T + _t[:, None]).astype(
    np.int32
)
BLKP = 56
_GIDX = np.zeros((NW, BLKP, RB), np.int32)
_GIDX[:, :BLKW, :] = (
    ((_b0[:, None] + np.arange(RB)[None, :]) * T + _t[:, None])
    .astype(np.int32)
    .reshape(NW, BLKW, RB)
)


def _sc_body(gidx_hbm, idx_hbm, table_hbm, mel_hbm, out_hbm,
             gidx_v, idx_v, rows_v, bufs, rsems, wsems, gat_sem):
    wid = lax.axis_index("s") * NC + lax.axis_index("c")
    m0 = wid * BLKW

    pltpu.sync_copy(idx_hbm.at[pl.ds(wid * BPW, BPW)], idx_v)
    pltpu.async_copy(table_hbm.at[idx_v], rows_v, gat_sem).wait()
    emb_wr = pltpu.async_copy(
        rows_v, out_hbm.at[pl.ds(wid * BPW, BPW), :], gat_sem
    )

    pltpu.sync_copy(gidx_hbm.at[wid], gidx_v)

    rd = [None] * BLKW
    wr = [None] * BLKW

    def start_read(i):
        return pltpu.async_copy(
            mel_hbm.at[gidx_v.at[i]], bufs[i % NBUF], rsems[i % NBUF]
        )

    for i in range(NBUF):
        rd[i] = start_read(i)
    for i in range(BLKW):
        rd[i].wait()
        m = m0 + i
        row0 = (1 + m // (B // RB)) * B + (m % (B // RB)) * RB
        wr[i] = pltpu.async_copy(
            bufs[i % NBUF], out_hbm.at[pl.ds(row0, RB), :], wsems[i % NBUF]
        )
        if i >= 1 and i + NBUF - 1 < BLKW:
            wr[i - 1].wait()
            rd[i + NBUF - 1] = start_read(i + NBUF - 1)
    for j in range(BLKW - NBUF, BLKW):
        wr[j].wait()
    emb_wr.wait()


@jax.jit
def _run(gidx, idx, table, mel2d):
    mesh = plsc.VectorSubcoreMesh(
        core_axis_name="c", subcore_axis_name="s", num_cores=NC, num_subcores=NS
    )
    out2d = pl.kernel(
        _sc_body,
        out_type=jax.ShapeDtypeStruct(((T + 1) * B, H), jnp.float32),
        mesh=mesh,
        scratch_types=[
            pltpu.VMEM((BLKP, RB), jnp.int32),
            pltpu.VMEM((BPW,), jnp.int32),
            pltpu.VMEM((BPW, H), jnp.float32),
            [pltpu.VMEM((RB, H), jnp.float32)] * NBUF,
            [pltpu.SemaphoreType.DMA] * NBUF,
            [pltpu.SemaphoreType.DMA] * NBUF,
            pltpu.SemaphoreType.DMA,
        ],
        compiler_params=pltpu.CompilerParams(use_tc_tiling_on_sc=True),
    )(gidx, idx, table, mel2d)
    return out2d.reshape(T + 1, B, H).transpose(1, 0, 2)


def kernel(arranger_id, mel_db, table):
    idx = arranger_id.reshape(B).astype(jnp.int32)
    gidx = jnp.asarray(_GIDX)
    return _run(gidx, idx, table, mel_db.reshape(B * T, H))
